# Initial kernel scaffold; baseline (speedup 1.0000x reference)
import jax
import jax.numpy as jnp
from jax.experimental import pallas as pl


def kernel(x, edge_index, batch, emb_table, W_ih, W_hh, W1_w, W2_w, W2_b, W3_w, q_w, q_b):
    raise NotImplementedError("write your pallas kernel here")


# speedup vs baseline: 3.6417x; 3.6417x over previous
import functools

import jax
import jax.numpy as jnp
from jax import lax
from jax.experimental import pallas as pl
from jax.experimental.pallas import tpu as pltpu
from jax.experimental.pallas import tpu_sc as plsc

H = 128
N_ITEMS = 100000
N_NODES = 10000
N_EDGES = 320000
N_SESS = 50

NPAD = 10240
SPAD = 64
NW = 32
CH = 80
EPW = 10080
EPAD = EPW * NW
NCHUNK = EPW // CH
ROWS_PW = NPAD // NW
STRIPE = NPAD // 16
RT = 1280
NT = NPAD // RT
VT = 2048
NVT = -(-N_ITEMS // VT)


def _sc_body(x_ref, src_ref, dst_ref, table_ref, zeros_ref,
             emb_out, m_out,
             x_v, idx_v, dst_v, rows_v, erows_v, m_sh, sem, sem2):
    cid = lax.axis_index("c")
    sid = lax.axis_index("s")
    wid = sid * 2 + cid

    pltpu.sync_copy(x_ref, x_v)
    pltpu.sync_copy(zeros_ref, m_sh.at[pl.ds(sid * STRIPE, STRIPE)])
    plsc.subcore_barrier()

    def emb_chunk(k, carry):
        off = wid * ROWS_PW + k * CH
        pltpu.async_copy(table_ref.at[x_v.at[pl.ds(off, CH)]], erows_v,
                         sem2).wait()
        pltpu.sync_copy(erows_v, emb_out.at[pl.ds(off, CH)])
        return carry

    lax.fori_loop(0, ROWS_PW // CH, emb_chunk, 0)

    def edge_chunk(c, carry):
        base = wid * EPW + c * CH
        pltpu.sync_copy(src_ref.at[pl.ds(base, CH)], idx_v)
        pltpu.sync_copy(dst_ref.at[pl.ds(base, CH)], dst_v)
        for j in range(CH // 16):
            s16 = idx_v[pl.ds(j * 16, 16)]
            idx_v[pl.ds(j * 16, 16)] = plsc.load_gather(x_v, [s16])
        pltpu.async_copy(table_ref.at[idx_v], rows_v, sem).wait()
        pltpu.sync_copy(rows_v, m_sh.at[dst_v], add=True)
        return carry

    lax.fori_loop(0, NCHUNK, edge_chunk, 0)
    plsc.subcore_barrier()

    pltpu.sync_copy(m_sh.at[pl.ds(sid * STRIPE, STRIPE)],
                    m_out.at[cid, pl.ds(sid * STRIPE, STRIPE)])


@functools.lru_cache(maxsize=1)
def _sc_graph():
    mesh = plsc.VectorSubcoreMesh(core_axis_name="c", subcore_axis_name="s")
    return pl.kernel(
        _sc_body,
        out_type=[
            jax.ShapeDtypeStruct((NPAD, H), jnp.float32),
            jax.ShapeDtypeStruct((2, NPAD, H), jnp.float32),
        ],
        mesh=mesh,
        scratch_types=[
            pltpu.VMEM((NPAD,), jnp.int32),
            pltpu.VMEM((CH,), jnp.int32),
            pltpu.VMEM((CH,), jnp.int32),
            pltpu.VMEM((CH, H), jnp.float32),
            pltpu.VMEM((CH, H), jnp.float32),
            pltpu.VMEM_SHARED((NPAD, H), jnp.float32),
            pltpu.SemaphoreType.DMA,
            pltpu.SemaphoreType.DMA,
        ],
        compiler_params=pltpu.CompilerParams(needs_layout_passes=False),
    )


def _gru_body(emb_ref, m0_ref, m1_ref, wih_ref, whh_ref, out_ref):
    emb = emb_ref[...]
    m = m0_ref[...] + m1_ref[...]
    dn = (((1,), (1,)), ((), ()))
    gi = lax.dot_general(m, wih_ref[...], dn,
                         preferred_element_type=jnp.float32)
    gh = lax.dot_general(emb, whh_ref[...], dn,
                         preferred_element_type=jnp.float32)
    r = jax.nn.sigmoid(gi[:, :H] + gh[:, :H])
    z = jax.nn.sigmoid(gi[:, H:2 * H] + gh[:, H:2 * H])
    n = jnp.tanh(gi[:, 2 * H:] + r * gh[:, 2 * H:])
    out_ref[...] = (1.0 - z) * n + z * emb


def _gru(emb, m0, m1, w_ih, w_hh):
    return pl.pallas_call(
        _gru_body,
        grid=(NT,),
        in_specs=[
            pl.BlockSpec((RT, H), lambda t: (t, 0)),
            pl.BlockSpec((RT, H), lambda t: (t, 0)),
            pl.BlockSpec((RT, H), lambda t: (t, 0)),
            pl.BlockSpec((3 * H, H), lambda t: (0, 0)),
            pl.BlockSpec((3 * H, H), lambda t: (0, 0)),
        ],
        out_specs=pl.BlockSpec((RT, H), lambda t: (t, 0)),
        out_shape=jax.ShapeDtypeStruct((NPAD, H), jnp.float32),
    )(emb, m0, m1, w_ih, w_hh)


def _vn_body(v_ref, b_ref, bn_ref, vn_out, cnt_out):
    t = pl.program_id(0)

    @pl.when(t == 0)
    def _():
        vn_out[...] = jnp.zeros_like(vn_out)
        cnt_out[...] = jnp.zeros_like(cnt_out)

    bt = b_ref[0]
    bnt = bn_ref[0]
    ioc = lax.broadcasted_iota(jnp.int32, (SPAD, RT), 0)
    oh_t = (ioc == bt).astype(jnp.float32)
    lastf = (bt != bnt).astype(jnp.float32)
    dn = (((1,), (0,)), ((), ()))
    vn_out[...] += lax.dot_general(oh_t * lastf, v_ref[...], dn,
                                   preferred_element_type=jnp.float32)
    cnt_out[...] += jnp.sum(oh_t, axis=1, keepdims=True)


def _vn(v_i, batch2, bnext2):
    return pl.pallas_call(
        _vn_body,
        grid=(NT,),
        in_specs=[
            pl.BlockSpec((RT, H), lambda t: (t, 0)),
            pl.BlockSpec((1, 1, RT), lambda t: (t, 0, 0)),
            pl.BlockSpec((1, 1, RT), lambda t: (t, 0, 0)),
        ],
        out_specs=[
            pl.BlockSpec((SPAD, H), lambda t: (0, 0)),
            pl.BlockSpec((SPAD, 1), lambda t: (0, 0)),
        ],
        out_shape=[
            jax.ShapeDtypeStruct((SPAD, H), jnp.float32),
            jax.ShapeDtypeStruct((SPAD, 1), jnp.float32),
        ],
    )(v_i, batch2, bnext2)


def _att_body(v_ref, b_ref, vn_ref, cnt_ref, w1_ref, w2_ref, w2b_ref,
              w3_ref, qw_ref, qb_ref, sh_out, vnf, sg):
    t = pl.program_id(0)

    @pl.when(t == 0)
    def _():
        v0 = v_ref[0:1, :]
        vnf[...] = jnp.where(cnt_ref[...] > 0.0, vn_ref[...], v0)
        sg[...] = jnp.zeros_like(sg)

    bt = b_ref[0]
    ioc = lax.broadcasted_iota(jnp.int32, (SPAD, RT), 0)
    oh_t = (ioc == bt).astype(jnp.float32)
    dnr = (((1,), (1,)), ((), ()))
    v = v_ref[...]
    vrep = lax.dot_general(oh_t, vnf[...], (((0,), (0,)), ((), ())),
                           preferred_element_type=jnp.float32)
    q1 = lax.dot_general(vrep, w1_ref[...], dnr,
                         preferred_element_type=jnp.float32)
    q2 = lax.dot_general(v, w2_ref[...], dnr,
                         preferred_element_type=jnp.float32) + w2b_ref[...]
    sig = jax.nn.sigmoid(q1 + q2)
    alpha = jnp.sum(sig * qw_ref[...], axis=1, keepdims=True) + qb_ref[...]
    sg[...] += lax.dot_general(oh_t, alpha * v, (((1,), (0,)), ((), ())),
                               preferred_element_type=jnp.float32)

    @pl.when(t == NT - 1)
    def _():
        w3 = w3_ref[...]
        sh_out[...] = (
            lax.dot_general(vnf[...], w3[:, :H], dnr,
                            preferred_element_type=jnp.float32)
            + lax.dot_general(sg[...], w3[:, H:], dnr,
                              preferred_element_type=jnp.float32))


def _att(v_i, batch2, vn, cnt, w1, w2, w2b, w3, qw, qb):
    return pl.pallas_call(
        _att_body,
        grid=(NT,),
        in_specs=[
            pl.BlockSpec((RT, H), lambda t: (t, 0)),
            pl.BlockSpec((1, 1, RT), lambda t: (t, 0, 0)),
            pl.BlockSpec((SPAD, H), lambda t: (0, 0)),
            pl.BlockSpec((SPAD, 1), lambda t: (0, 0)),
            pl.BlockSpec((H, H), lambda t: (0, 0)),
            pl.BlockSpec((H, H), lambda t: (0, 0)),
            pl.BlockSpec((1, H), lambda t: (0, 0)),
            pl.BlockSpec((H, 2 * H), lambda t: (0, 0)),
            pl.BlockSpec((1, H), lambda t: (0, 0)),
            pl.BlockSpec((1, 1), lambda t: (0, 0)),
        ],
        out_specs=pl.BlockSpec((SPAD, H), lambda t: (0, 0)),
        out_shape=jax.ShapeDtypeStruct((SPAD, H), jnp.float32),
        scratch_shapes=[
            pltpu.VMEM((SPAD, H), jnp.float32),
            pltpu.VMEM((SPAD, H), jnp.float32),
        ],
    )(v_i, batch2, vn, cnt, w1, w2, w2b, w3, qw, qb)


def _readout_body(sh_ref, tab_ref, out_ref):
    out_ref[...] = lax.dot_general(sh_ref[...], tab_ref[...],
                                   (((1,), (1,)), ((), ())),
                                   preferred_element_type=jnp.float32)


def _readout(sh, table):
    return pl.pallas_call(
        _readout_body,
        grid=(NVT,),
        in_specs=[
            pl.BlockSpec((N_SESS, H), lambda t: (0, 0)),
            pl.BlockSpec((VT, H), lambda t: (t, 0)),
        ],
        out_specs=pl.BlockSpec((N_SESS, VT), lambda t: (0, t)),
        out_shape=jax.ShapeDtypeStruct((N_SESS, N_ITEMS), jnp.float32),
    )(sh, table)


def kernel(x, edge_index, batch, emb_table, W_ih, W_hh, W1_w, W2_w, W2_b,
           W3_w, q_w, q_b):
    i32 = jnp.int32
    f32 = jnp.float32
    x = x.astype(i32)
    src = edge_index[0].astype(i32)
    dst = edge_index[1].astype(i32)
    batch = batch.astype(i32)

    x_pad = jnp.concatenate([x, jnp.zeros((NPAD - N_NODES,), i32)])
    src_pad = jnp.concatenate([src, jnp.zeros((EPAD - N_EDGES,), i32)])
    dst_pad = jnp.concatenate(
        [dst, jnp.full((EPAD - N_EDGES,), NPAD - 1, i32)])
    zeros = jnp.zeros((STRIPE, H), f32)

    emb_pad, m_parts = _sc_graph()(x_pad, src_pad, dst_pad,
                                   emb_table.astype(f32), zeros)

    v_i = _gru(emb_pad, m_parts[0], m_parts[1], W_ih, W_hh)

    batch_pad = jnp.concatenate(
        [batch, jnp.full((NPAD - N_NODES,), SPAD - 1, i32)])
    bnext = jnp.concatenate([batch_pad[1:], jnp.full((1,), 1 << 20, i32)])
    batch2 = batch_pad.reshape(NT, 1, RT)
    bnext2 = bnext.reshape(NT, 1, RT)

    vn, cnt = _vn(v_i, batch2, bnext2)
    sh = _att(v_i, batch2, vn, cnt, W1_w, W2_w,
              W2_b.reshape(1, H), W3_w, q_w.reshape(1, H),
              q_b.reshape(1, 1))

    return _readout(sh[:N_SESS], emb_table.astype(f32))
